# baseline (speedup 1.0000x reference)
import functools

import jax
import jax.numpy as jnp
from jax import lax
from jax.experimental import pallas as pl
from jax.experimental.pallas import tpu as pltpu
from jax.experimental.pallas import tpu_sc as plsc

B = 4096
L = 200
EMB_DIM = 128
VOCAB = 1000

NC = 2
NS = 16
LANES = 16
NW = NC * NS
ROWS_PER_W = B // NW
GROUPS = ROWS_PER_W // LANES


def _proj_body(emb_ref, wcat_ref, bhalf_ref, out_ref):
    out_ref[...] = jnp.dot(emb_ref[...], wcat_ref[...],
                           preferred_element_type=jnp.float32) + bhalf_ref[...]


_proj = pl.pallas_call(
    _proj_body,
    out_shape=jax.ShapeDtypeStruct((VOCAB, 4), jnp.float32),
)


_sc_mesh = plsc.VectorSubcoreMesh(core_axis_name="c", subcore_axis_name="s")


@functools.partial(
    pl.kernel,
    out_type=jax.ShapeDtypeStruct((2 * B,), jnp.float32),
    mesh=_sc_mesh,
    compiler_params=pltpu.CompilerParams(needs_layout_passes=False),
    scratch_types=[
        pltpu.VMEM((4 * VOCAB,), jnp.float32),
        pltpu.VMEM((ROWS_PER_W * L,), jnp.int32),
        pltpu.VMEM((ROWS_PER_W * L,), jnp.int32),
        pltpu.VMEM((2 * ROWS_PER_W,), jnp.float32),
    ],
)
def _sc_gather(table_hbm, idx1_hbm, idx2_hbm, out_hbm,
               table_v, idx1_v, idx2_v, out_v):
    wid = lax.axis_index("s") * NC + lax.axis_index("c")
    base = wid * ROWS_PER_W
    pltpu.sync_copy(table_hbm, table_v)
    pltpu.sync_copy(idx1_hbm.at[pl.ds(base * L, ROWS_PER_W * L)], idx1_v)
    pltpu.sync_copy(idx2_hbm.at[pl.ds(base * L, ROWS_PER_W * L)], idx2_v)
    for g in range(GROUPS):
        lrow = lax.iota(jnp.int32, LANES) + g * LANES
        row_off = lrow * L
        zero = jnp.zeros((LANES,), jnp.float32)

        def step(l, carry, row_off=row_off):
            a0, a1 = carry
            col = jnp.full((LANES,), l, jnp.int32)
            vA = plsc.load_gather(idx1_v, [row_off + col])
            vB = plsc.load_gather(idx2_v, [row_off + col])
            pA = vA * 4
            pB = vB * 4 + 2
            g0 = plsc.load_gather(table_v, [pA]) + plsc.load_gather(table_v, [pB])
            g1 = plsc.load_gather(table_v, [pA + 1]) + plsc.load_gather(table_v, [pB + 1])
            return a0 + g0, a1 + g1

        a0, a1 = plsc.parallel_loop(0, L, unroll=8, carry=(zero, zero))(step)

        l0 = a0 * (1.0 / L)
        l1 = a1 * (1.0 / L)
        act0 = jnp.where(l0 >= 0, l0, 0.01 * l0)
        act1 = jnp.where(l1 >= 0, l1, 0.01 * l1)
        m = jnp.maximum(act0, act1)
        d0 = act0 - m
        d1 = act1 - m
        z = jnp.exp(d0) + jnp.exp(d1)
        w = (z - 1.0) / (z + 1.0)
        w2 = w * w
        logz = w * (2.0 + w2 * (2.0 / 3.0 + w2 * (2.0 / 5.0 + w2 * (2.0 / 7.0))))
        plsc.store_scatter(out_v, [lrow * 2], d0 - logz)
        plsc.store_scatter(out_v, [lrow * 2 + 1], d1 - logz)
    pltpu.sync_copy(out_v, out_hbm.at[pl.ds(base * 2, 2 * ROWS_PER_W)])


def kernel(input_ch1, input_ch2, emb, W, b):
    wcat = jnp.concatenate([W[:, :EMB_DIM].T, W[:, EMB_DIM:].T], axis=1)
    bhalf = (0.5 * jnp.concatenate([b, b])).reshape(1, 4)
    t = _proj(emb, wcat, bhalf).reshape(-1)
    out = _sc_gather(t, input_ch1.reshape(-1), input_ch2.reshape(-1))
    return out.reshape(B, 2)


# speedup vs baseline: 1.4670x; 1.4670x over previous
import functools

import jax
import jax.numpy as jnp
from jax import lax
from jax.experimental import pallas as pl
from jax.experimental.pallas import tpu as pltpu
from jax.experimental.pallas import tpu_sc as plsc

B = 4096
L = 200
EMB_DIM = 128
VOCAB = 1000
TBL = 2048

NC = 2
NS = 16
LANES = 16
NW = NC * NS
COLS_PER_W = B // NW
GROUPS = COLS_PER_W // LANES
LT = L // 8
LT_A = 13


def _pack_pair(t2):
    lo = lax.bitcast_convert_type(t2[0:1, :].astype(jnp.bfloat16), jnp.uint16)
    hi = lax.bitcast_convert_type(t2[1:2, :].astype(jnp.bfloat16), jnp.uint16)
    packed = lo.astype(jnp.uint32) | (hi.astype(jnp.uint32) << 16)
    return lax.bitcast_convert_type(packed, jnp.int32)


def _proj_body(emb_ref, w_ref, b2_ref, out_ref):
    emb = emb_ref[...]
    bh = 0.5 * b2_ref[...]
    dn = (((1,), (1,)), ((), ()))
    tA = lax.dot_general(w_ref[:, :EMB_DIM], emb, dn,
                         preferred_element_type=jnp.float32) + bh.T
    tB = lax.dot_general(w_ref[:, EMB_DIM:], emb, dn,
                         preferred_element_type=jnp.float32) + bh.T
    out_ref[pl.ds(0, VOCAB)] = _pack_pair(tA).reshape(VOCAB)
    out_ref[pl.ds(1024, VOCAB)] = _pack_pair(tB).reshape(VOCAB)


_proj = pl.pallas_call(
    _proj_body,
    out_shape=jax.ShapeDtypeStruct((TBL,), jnp.int32),
)


_sc_mesh = plsc.VectorSubcoreMesh(core_axis_name="c", subcore_axis_name="s")


@functools.partial(
    pl.kernel,
    out_type=jax.ShapeDtypeStruct((2 * B,), jnp.float32),
    mesh=_sc_mesh,
    compiler_params=pltpu.CompilerParams(needs_layout_passes=False),
    scratch_types=[
        pltpu.VMEM((TBL,), jnp.int32),
        pltpu.VMEM((LT, 8, COLS_PER_W), jnp.int32),
        pltpu.VMEM((LT, 8, COLS_PER_W), jnp.int32),
        pltpu.VMEM((2 * COLS_PER_W,), jnp.float32),
        pltpu.VMEM((2 * COLS_PER_W,), jnp.float32),
        pltpu.SemaphoreType.DMA,
        pltpu.SemaphoreType.DMA,
        pltpu.SemaphoreType.DMA,
        pltpu.SemaphoreType.DMA,
    ],
)
def _sc_gather(table_hbm, idx1_hbm, idx2_hbm, out_hbm,
               table_v, idx1_v, idx2_v, acc_v, out_v,
               sem_t, sem_1a, sem_1b, sem_2):
    wid = lax.axis_index("s") * NC + lax.axis_index("c")
    base = wid * COLS_PER_W
    d_t = pltpu.async_copy(table_hbm, table_v, sem_t)
    d_1a = pltpu.async_copy(
        idx1_hbm.at[pl.ds(0, LT_A), :, pl.ds(base, COLS_PER_W)],
        idx1_v.at[pl.ds(0, LT_A)], sem_1a)
    d_1b = pltpu.async_copy(
        idx1_hbm.at[pl.ds(LT_A, LT - LT_A), :, pl.ds(base, COLS_PER_W)],
        idx1_v.at[pl.ds(LT_A, LT - LT_A)], sem_1b)
    d_2 = pltpu.async_copy(idx2_hbm.at[:, :, pl.ds(base, COLS_PER_W)],
                           idx2_v, sem_2)
    himask = jnp.full((LANES,), -65536, jnp.int32)
    zero = jnp.zeros((LANES,), jnp.float32)
    for g in range(2 * GROUPS):
        acc_v[pl.ds(g * LANES, LANES)] = zero

    def accum(idx_v, off):
        def body(lt):
            for lm in range(8):
                for g in range(GROUPS):
                    v = idx_v[lt, lm, pl.ds(g * LANES, LANES)]
                    gA = plsc.load_gather(table_v, [v + off] if off else [v])
                    plsc.addupdate(acc_v.at[pl.ds(g * LANES, LANES)],
                                   plsc.bitcast(gA << 16, jnp.float32))
                    plsc.addupdate(
                        acc_v.at[pl.ds(COLS_PER_W + g * LANES, LANES)],
                        plsc.bitcast(gA & himask, jnp.float32))
        return body

    d_t.wait()
    d_1a.wait()
    plsc.parallel_loop(0, LT_A)(accum(idx1_v, 0))
    d_1b.wait()
    plsc.parallel_loop(LT_A, LT)(accum(idx1_v, 0))
    d_2.wait()
    plsc.parallel_loop(0, LT)(accum(idx2_v, 1024))

    for g in range(GROUPS):
        a0 = acc_v[pl.ds(g * LANES, LANES)]
        a1 = acc_v[pl.ds(COLS_PER_W + g * LANES, LANES)]
        l0 = a0 * (1.0 / L)
        l1 = a1 * (1.0 / L)
        act0 = jnp.where(l0 >= 0, l0, 0.01 * l0)
        act1 = jnp.where(l1 >= 0, l1, 0.01 * l1)
        m = jnp.maximum(act0, act1)
        d0 = act0 - m
        d1 = act1 - m
        z = jnp.exp(d0) + jnp.exp(d1)
        w = (z - 1.0) / (z + 1.0)
        w2 = w * w
        logz = w * (2.0 + w2 * (2.0 / 3.0 + w2 * (2.0 / 5.0 + w2 * (2.0 / 7.0))))
        out_v[pl.ds(g * LANES, LANES)] = d0 - logz
        out_v[pl.ds(COLS_PER_W + g * LANES, LANES)] = d1 - logz

    pltpu.sync_copy(out_v.at[pl.ds(0, COLS_PER_W)],
                    out_hbm.at[pl.ds(base, COLS_PER_W)])
    pltpu.sync_copy(out_v.at[pl.ds(COLS_PER_W, COLS_PER_W)],
                    out_hbm.at[pl.ds(B + base, COLS_PER_W)])


def kernel(input_ch1, input_ch2, emb, W, b):
    t = _proj(emb, W, b.reshape(1, 2))
    i1 = input_ch1.T.reshape(LT, 8, B)
    i2 = input_ch2.T.reshape(LT, 8, B)
    out = _sc_gather(t, i1, i2)
    return out.reshape(2, B).T


# speedup vs baseline: 1.9394x; 1.3220x over previous
import functools

import jax
import jax.numpy as jnp
from jax import lax
from jax.experimental import pallas as pl
from jax.experimental.pallas import tpu as pltpu
from jax.experimental.pallas import tpu_sc as plsc

B = 4096
L = 200
EMB_DIM = 128
VOCAB = 1000
TBL = 2048

NC = 2
NS = 16
LANES = 16
NW = NC * NS
COLS_PER_W = B // NW
GROUPS = COLS_PER_W // LANES
LT = L // 8
LT_A = 13


def _pack_pair(t2):
    lo = lax.bitcast_convert_type(t2[0:1, :].astype(jnp.bfloat16), jnp.uint16)
    hi = lax.bitcast_convert_type(t2[1:2, :].astype(jnp.bfloat16), jnp.uint16)
    packed = lo.astype(jnp.uint32) | (hi.astype(jnp.uint32) << 16)
    return lax.bitcast_convert_type(packed, jnp.int32)


def _proj_body(emb_ref, w_ref, b2_ref, out_ref):
    emb = emb_ref[...]
    bh = 0.5 * b2_ref[...]
    dn = (((1,), (1,)), ((), ()))
    tA = lax.dot_general(w_ref[:, :EMB_DIM], emb, dn,
                         preferred_element_type=jnp.float32) + bh.T
    tB = lax.dot_general(w_ref[:, EMB_DIM:], emb, dn,
                         preferred_element_type=jnp.float32) + bh.T
    out_ref[pl.ds(0, VOCAB)] = _pack_pair(tA).reshape(VOCAB)
    out_ref[pl.ds(1024, VOCAB)] = _pack_pair(tB).reshape(VOCAB)


_proj = pl.pallas_call(
    _proj_body,
    out_shape=jax.ShapeDtypeStruct((TBL,), jnp.int32),
)


_sc_mesh = plsc.VectorSubcoreMesh(core_axis_name="c", subcore_axis_name="s")


@functools.partial(
    pl.kernel,
    out_type=jax.ShapeDtypeStruct((2 * B,), jnp.float32),
    mesh=_sc_mesh,
    compiler_params=pltpu.CompilerParams(needs_layout_passes=False),
    scratch_types=[
        pltpu.VMEM((TBL,), jnp.int32),
        pltpu.VMEM((LT, 8, COLS_PER_W), jnp.int32),
        pltpu.VMEM((LT, 8, COLS_PER_W), jnp.int32),
        pltpu.VMEM((2 * COLS_PER_W,), jnp.float32),
        pltpu.VMEM((2 * COLS_PER_W,), jnp.float32),
        pltpu.SemaphoreType.DMA,
        pltpu.SemaphoreType.DMA,
        pltpu.SemaphoreType.DMA,
        pltpu.SemaphoreType.DMA,
    ],
)
def _sc_gather(table_hbm, idx1_hbm, idx2_hbm, out_hbm,
               table_v, idx1_v, idx2_v, acc_v, out_v,
               sem_t, sem_1a, sem_1b, sem_2):
    wid = lax.axis_index("s") * NC + lax.axis_index("c")
    base = wid * COLS_PER_W
    d_t = pltpu.async_copy(table_hbm, table_v, sem_t)
    d_1a = pltpu.async_copy(
        idx1_hbm.at[pl.ds(0, LT_A), :, pl.ds(base, COLS_PER_W)],
        idx1_v.at[pl.ds(0, LT_A)], sem_1a)
    d_1b = pltpu.async_copy(
        idx1_hbm.at[pl.ds(LT_A, LT - LT_A), :, pl.ds(base, COLS_PER_W)],
        idx1_v.at[pl.ds(LT_A, LT - LT_A)], sem_1b)
    d_2 = pltpu.async_copy(idx2_hbm.at[:, :, pl.ds(base, COLS_PER_W)],
                           idx2_v, sem_2)
    himask = jnp.full((LANES,), -65536, jnp.int32)
    zeros = tuple(jnp.zeros((LANES,), jnp.float32) for _ in range(2 * GROUPS))

    def accum(idx_v, off):
        def body(lt, carry):
            acc = list(carry)
            for lm in range(8):
                for g in range(GROUPS):
                    v = idx_v[lt, lm, pl.ds(g * LANES, LANES)]
                    gX = plsc.load_gather(table_v, [v + off] if off else [v])
                    acc[2 * g] = acc[2 * g] + plsc.bitcast(gX << 16, jnp.float32)
                    acc[2 * g + 1] = acc[2 * g + 1] + plsc.bitcast(
                        gX & himask, jnp.float32)
            return tuple(acc)
        return body

    d_t.wait()
    d_1a.wait()
    acc = plsc.parallel_loop(0, LT_A, carry=zeros)(accum(idx1_v, 0))
    d_1b.wait()
    acc = plsc.parallel_loop(LT_A, LT, carry=acc)(accum(idx1_v, 0))
    d_2.wait()
    acc = plsc.parallel_loop(0, LT, carry=acc)(accum(idx2_v, 1024))

    for g in range(GROUPS):
        a0 = acc[2 * g]
        a1 = acc[2 * g + 1]

        l0 = a0 * (1.0 / L)
        l1 = a1 * (1.0 / L)
        act0 = jnp.where(l0 >= 0, l0, 0.01 * l0)
        act1 = jnp.where(l1 >= 0, l1, 0.01 * l1)
        m = jnp.maximum(act0, act1)
        d0 = act0 - m
        d1 = act1 - m
        z = jnp.exp(d0) + jnp.exp(d1)
        w = (z - 1.0) / (z + 1.0)
        w2 = w * w
        logz = w * (2.0 + w2 * (2.0 / 3.0 + w2 * (2.0 / 5.0 + w2 * (2.0 / 7.0))))
        out_v[pl.ds(g * LANES, LANES)] = d0 - logz
        out_v[pl.ds(COLS_PER_W + g * LANES, LANES)] = d1 - logz

    pltpu.sync_copy(out_v.at[pl.ds(0, COLS_PER_W)],
                    out_hbm.at[pl.ds(base, COLS_PER_W)])
    pltpu.sync_copy(out_v.at[pl.ds(COLS_PER_W, COLS_PER_W)],
                    out_hbm.at[pl.ds(B + base, COLS_PER_W)])


def kernel(input_ch1, input_ch2, emb, W, b):
    t = _proj(emb, W, b.reshape(1, 2))
    i1 = input_ch1.T.reshape(LT, 8, B)
    i2 = input_ch2.T.reshape(LT, 8, B)
    out = _sc_gather(t, i1, i2)
    return out.reshape(2, B).T


# speedup vs baseline: 2.0278x; 1.0456x over previous
import functools

import jax
import jax.numpy as jnp
from jax import lax
from jax.experimental import pallas as pl
from jax.experimental.pallas import tpu as pltpu
from jax.experimental.pallas import tpu_sc as plsc

B = 4096
L = 200
EMB_DIM = 128
VOCAB = 1000
TBL = 2048

NC = 2
NS = 16
LANES = 16
NW = NC * NS
COLS_PER_W = B // NW
GROUPS = COLS_PER_W // LANES
LT = L // 8


def _pack_pair(t2):
    lo = lax.bitcast_convert_type(t2[0:1, :].astype(jnp.bfloat16), jnp.uint16)
    hi = lax.bitcast_convert_type(t2[1:2, :].astype(jnp.bfloat16), jnp.uint16)
    packed = lo.astype(jnp.uint32) | (hi.astype(jnp.uint32) << 16)
    return lax.bitcast_convert_type(packed, jnp.int32)


def _proj_body(emb_ref, w_ref, b2_ref, out_ref):
    emb = emb_ref[...]
    bh = 0.5 * b2_ref[...]
    dn = (((1,), (1,)), ((), ()))
    tA = lax.dot_general(w_ref[:, :EMB_DIM], emb, dn,
                         preferred_element_type=jnp.float32) + bh.T
    tB = lax.dot_general(w_ref[:, EMB_DIM:], emb, dn,
                         preferred_element_type=jnp.float32) + bh.T
    out_ref[pl.ds(0, VOCAB)] = _pack_pair(tA).reshape(VOCAB)
    out_ref[pl.ds(1024, VOCAB)] = _pack_pair(tB).reshape(VOCAB)


_proj = pl.pallas_call(
    _proj_body,
    out_shape=jax.ShapeDtypeStruct((TBL,), jnp.int32),
)


_sc_mesh = plsc.VectorSubcoreMesh(core_axis_name="c", subcore_axis_name="s")


@functools.partial(
    pl.kernel,
    out_type=jax.ShapeDtypeStruct((2 * B,), jnp.float32),
    mesh=_sc_mesh,
    compiler_params=pltpu.CompilerParams(needs_layout_passes=False),
    scratch_types=[
        pltpu.VMEM((TBL,), jnp.int32),
        pltpu.VMEM((LT, 8, COLS_PER_W), jnp.int32),
        pltpu.VMEM((LT, 8, COLS_PER_W), jnp.int32),
        pltpu.VMEM((2 * COLS_PER_W,), jnp.float32),
        pltpu.VMEM((2 * COLS_PER_W,), jnp.float32),
        pltpu.SemaphoreType.DMA,
        pltpu.SemaphoreType.DMA,
        pltpu.SemaphoreType.DMA,
    ],
)
def _sc_gather(table_hbm, idx1_hbm, idx2_hbm, out_hbm,
               table_v, idx1_v, idx2_v, acc_v, out_v, sem_t, sem_1, sem_2):
    wid = lax.axis_index("s") * NC + lax.axis_index("c")
    base = wid * COLS_PER_W
    d_t = pltpu.async_copy(table_hbm, table_v, sem_t)
    d_1 = pltpu.async_copy(idx1_hbm.at[:, :, pl.ds(base, COLS_PER_W)],
                           idx1_v, sem_1)
    d_2 = pltpu.async_copy(idx2_hbm.at[:, :, pl.ds(base, COLS_PER_W)],
                           idx2_v, sem_2)
    d_t.wait()
    d_1.wait()
    himask = jnp.full((LANES,), -65536, jnp.int32)
    zero = jnp.zeros((LANES,), jnp.float32)

    zeros8 = (zero,) * 8

    def make_step(idx_v, off):
        def step(lt, carry, ):
            acc = list(carry)
            for lm in range(8):
                gX = plsc.load_gather(
                    table_v,
                    [idx_v[lt, lm, pl.ds(step.g * LANES, LANES)] + off]
                    if off else [idx_v[lt, lm, pl.ds(step.g * LANES, LANES)]])
                k = lm % 4
                acc[2 * k] = acc[2 * k] + plsc.bitcast(gX << 16, jnp.float32)
                acc[2 * k + 1] = acc[2 * k + 1] + plsc.bitcast(
                    gX & himask, jnp.float32)
            return tuple(acc)
        return step

    for g in range(GROUPS):
        step1 = make_step(idx1_v, 0)
        step1.g = g
        acc = plsc.parallel_loop(0, LT, unroll=2, carry=zeros8)(step1)
        acc_v[pl.ds(g * LANES, LANES)] = (acc[0] + acc[2]) + (acc[4] + acc[6])
        acc_v[pl.ds(COLS_PER_W + g * LANES, LANES)] = (
            (acc[1] + acc[3]) + (acc[5] + acc[7]))

    d_2.wait()

    for g in range(GROUPS):
        step2 = make_step(idx2_v, 1024)
        step2.g = g
        acc = plsc.parallel_loop(0, LT, unroll=2, carry=zeros8)(step2)
        a0 = ((acc[0] + acc[2]) + (acc[4] + acc[6])
              + acc_v[pl.ds(g * LANES, LANES)])
        a1 = ((acc[1] + acc[3]) + (acc[5] + acc[7])
              + acc_v[pl.ds(COLS_PER_W + g * LANES, LANES)])

        l0 = a0 * (1.0 / L)
        l1 = a1 * (1.0 / L)
        act0 = jnp.where(l0 >= 0, l0, 0.01 * l0)
        act1 = jnp.where(l1 >= 0, l1, 0.01 * l1)
        m = jnp.maximum(act0, act1)
        d0 = act0 - m
        d1 = act1 - m
        z = jnp.exp(d0) + jnp.exp(d1)
        w = (z - 1.0) / (z + 1.0)
        w2 = w * w
        logz = w * (2.0 + w2 * (2.0 / 3.0 + w2 * (2.0 / 5.0 + w2 * (2.0 / 7.0))))
        out_v[pl.ds(g * LANES, LANES)] = d0 - logz
        out_v[pl.ds(COLS_PER_W + g * LANES, LANES)] = d1 - logz

    pltpu.sync_copy(out_v.at[pl.ds(0, COLS_PER_W)],
                    out_hbm.at[pl.ds(base, COLS_PER_W)])
    pltpu.sync_copy(out_v.at[pl.ds(COLS_PER_W, COLS_PER_W)],
                    out_hbm.at[pl.ds(B + base, COLS_PER_W)])


def kernel(input_ch1, input_ch2, emb, W, b):
    t = _proj(emb, W, b.reshape(1, 2))
    i1 = input_ch1.T.reshape(LT, 8, B)
    i2 = input_ch2.T.reshape(LT, 8, B)
    out = _sc_gather(t, i1, i2)
    return out.reshape(2, B).T


# speedup vs baseline: 2.0325x; 1.0023x over previous
import functools

import jax
import jax.numpy as jnp
from jax import lax
from jax.experimental import pallas as pl
from jax.experimental.pallas import tpu as pltpu
from jax.experimental.pallas import tpu_sc as plsc

B = 4096
L = 200
EMB_DIM = 128
VOCAB = 1000
TBL = 2048

NC = 2
NS = 16
LANES = 16
NW = NC * NS
COLS_PER_W = B // NW
GROUPS = COLS_PER_W // LANES
LT = L // 8


def _pack_pair(t2):
    lo = lax.bitcast_convert_type(t2[0:1, :].astype(jnp.bfloat16), jnp.uint16)
    hi = lax.bitcast_convert_type(t2[1:2, :].astype(jnp.bfloat16), jnp.uint16)
    packed = lo.astype(jnp.uint32) | (hi.astype(jnp.uint32) << 16)
    return lax.bitcast_convert_type(packed, jnp.int32)


def _proj_body(emb_ref, w_ref, b2_ref, out_ref):
    emb = emb_ref[...]
    bh = 0.5 * b2_ref[...]
    dn = (((1,), (1,)), ((), ()))
    tA = lax.dot_general(w_ref[:, :EMB_DIM], emb, dn,
                         preferred_element_type=jnp.float32) + bh.T
    tB = lax.dot_general(w_ref[:, EMB_DIM:], emb, dn,
                         preferred_element_type=jnp.float32) + bh.T
    out_ref[pl.ds(0, VOCAB)] = _pack_pair(tA).reshape(VOCAB)
    out_ref[pl.ds(1024, VOCAB)] = _pack_pair(tB).reshape(VOCAB)


_proj = pl.pallas_call(
    _proj_body,
    out_shape=jax.ShapeDtypeStruct((TBL,), jnp.int32),
)


_sc_mesh = plsc.VectorSubcoreMesh(core_axis_name="c", subcore_axis_name="s")


@functools.partial(
    pl.kernel,
    out_type=jax.ShapeDtypeStruct((2 * B,), jnp.float32),
    mesh=_sc_mesh,
    compiler_params=pltpu.CompilerParams(needs_layout_passes=False,
                                         skip_device_barrier=True),
    scratch_types=[
        pltpu.VMEM((TBL,), jnp.int32),
        pltpu.VMEM((LT, 8, COLS_PER_W), jnp.int32),
        pltpu.VMEM((LT, 8, COLS_PER_W), jnp.int32),
        pltpu.VMEM((2 * COLS_PER_W,), jnp.float32),
        pltpu.VMEM((2 * COLS_PER_W,), jnp.float32),
        pltpu.SemaphoreType.DMA,
        pltpu.SemaphoreType.DMA,
        pltpu.SemaphoreType.DMA,
    ],
)
def _sc_gather(table_hbm, idx1_hbm, idx2_hbm, out_hbm,
               table_v, idx1_v, idx2_v, acc_v, out_v, sem_t, sem_1, sem_2):
    wid = lax.axis_index("s") * NC + lax.axis_index("c")
    base = wid * COLS_PER_W
    d_t = pltpu.async_copy(table_hbm, table_v, sem_t)
    d_1 = pltpu.async_copy(idx1_hbm.at[:, :, pl.ds(base, COLS_PER_W)],
                           idx1_v, sem_1)
    d_2 = pltpu.async_copy(idx2_hbm.at[:, :, pl.ds(base, COLS_PER_W)],
                           idx2_v, sem_2)
    d_t.wait()
    d_1.wait()
    himask = jnp.full((LANES,), -65536, jnp.int32)
    zero = jnp.zeros((LANES,), jnp.float32)

    zeros8 = (zero,) * 8

    def make_step(idx_v, off):
        def step(lt, carry, ):
            acc = list(carry)
            for lm in range(8):
                gX = plsc.load_gather(
                    table_v,
                    [idx_v[lt, lm, pl.ds(step.g * LANES, LANES)] + off]
                    if off else [idx_v[lt, lm, pl.ds(step.g * LANES, LANES)]])
                k = lm % 4
                acc[2 * k] = acc[2 * k] + plsc.bitcast(gX << 16, jnp.float32)
                acc[2 * k + 1] = acc[2 * k + 1] + plsc.bitcast(
                    gX & himask, jnp.float32)
            return tuple(acc)
        return step

    for g in range(GROUPS):
        step1 = make_step(idx1_v, 0)
        step1.g = g
        acc = plsc.parallel_loop(0, LT, unroll=2, carry=zeros8)(step1)
        acc_v[pl.ds(g * LANES, LANES)] = (acc[0] + acc[2]) + (acc[4] + acc[6])
        acc_v[pl.ds(COLS_PER_W + g * LANES, LANES)] = (
            (acc[1] + acc[3]) + (acc[5] + acc[7]))

    d_2.wait()

    for g in range(GROUPS):
        step2 = make_step(idx2_v, 1024)
        step2.g = g
        acc = plsc.parallel_loop(0, LT, unroll=2, carry=zeros8)(step2)
        a0 = ((acc[0] + acc[2]) + (acc[4] + acc[6])
              + acc_v[pl.ds(g * LANES, LANES)])
        a1 = ((acc[1] + acc[3]) + (acc[5] + acc[7])
              + acc_v[pl.ds(COLS_PER_W + g * LANES, LANES)])

        l0 = a0 * (1.0 / L)
        l1 = a1 * (1.0 / L)
        act0 = jnp.where(l0 >= 0, l0, 0.01 * l0)
        act1 = jnp.where(l1 >= 0, l1, 0.01 * l1)
        m = jnp.maximum(act0, act1)
        d0 = act0 - m
        d1 = act1 - m
        z = jnp.exp(d0) + jnp.exp(d1)
        w = (z - 1.0) / (z + 1.0)
        w2 = w * w
        logz = w * (2.0 + w2 * (2.0 / 3.0 + w2 * (2.0 / 5.0 + w2 * (2.0 / 7.0))))
        out_v[pl.ds(g * LANES, LANES)] = d0 - logz
        out_v[pl.ds(COLS_PER_W + g * LANES, LANES)] = d1 - logz

    pltpu.sync_copy(out_v.at[pl.ds(0, COLS_PER_W)],
                    out_hbm.at[pl.ds(base, COLS_PER_W)])
    pltpu.sync_copy(out_v.at[pl.ds(COLS_PER_W, COLS_PER_W)],
                    out_hbm.at[pl.ds(B + base, COLS_PER_W)])


def kernel(input_ch1, input_ch2, emb, W, b):
    t = _proj(emb, W, b.reshape(1, 2))
    i1 = input_ch1.T.reshape(LT, 8, B)
    i2 = input_ch2.T.reshape(LT, 8, B)
    out = _sc_gather(t, i1, i2)
    return out.reshape(2, B).T
